# Initial kernel scaffold; baseline (speedup 1.0000x reference)
import jax
import jax.numpy as jnp
from jax.experimental import pallas as pl


def kernel(x, edge_index, W_in, b_in, Wc, bc, gamma, beta, W_out, b_out):
    raise NotImplementedError("write your pallas kernel here")


# speedup vs baseline: 11.8886x; 11.8886x over previous
import dataclasses
import functools

import jax
import jax.numpy as jnp
from jax import lax
from jax.experimental import pallas as pl
from jax.experimental.pallas import tpu as pltpu
from jax.experimental.pallas import tpu_sc as plsc

N = 10000
E = 320000
D = 128
H = 128
O_DIM = 10
L = 3
EPS = 1e-5

NC = 2
NS = 16
NW = NC * NS
EW = E // NW
CHUNK = 80
STAGE = 2000
N_PAD = 10112
ROWS_PER_TILE = N_PAD // NS
ZR = 8

_mesh = functools.partial(
    plsc.VectorSubcoreMesh, core_axis_name="c", subcore_axis_name="s"
)


def _sc_params():
    cp = pltpu.CompilerParams()
    if "needs_layout_passes" in pltpu.CompilerParams.__dataclass_fields__:
        cp = dataclasses.replace(cp, needs_layout_passes=False)
    return cp


def _deg_body(dst_hbm, out_hbm, idx_v, deg_v, sem):
    c = lax.axis_index("c")
    s = lax.axis_index("s")
    wid = s * NC + c

    zeros16 = jnp.zeros((16,), jnp.float32)
    ones16 = jnp.ones((16,), jnp.float32)

    @pl.loop(0, N, step=16)
    def _(i):
        deg_v[pl.ds(i, 16)] = zeros16

    base = wid * EW

    @pl.loop(0, EW, step=STAGE)
    def _(j):
        pltpu.async_copy(dst_hbm.at[pl.ds(base + j, STAGE)], idx_v, sem).wait()

        @pl.loop(0, STAGE, step=16)
        def _(k):
            idx = idx_v[pl.ds(k, 16)]
            plsc.addupdate_scatter(deg_v, [idx], ones16)

    pltpu.sync_copy(deg_v, out_hbm.at[pl.ds(wid * N, N)])


def _sc_degree(dst):
    k = pl.kernel(
        _deg_body,
        out_type=jax.ShapeDtypeStruct((NW * N,), jnp.float32),
        mesh=_mesh(),
        scratch_types=[
            pltpu.VMEM((STAGE,), jnp.int32),
            pltpu.VMEM((N,), jnp.float32),
            pltpu.SemaphoreType.DMA,
        ],
        compiler_params=_sc_params(),
    )
    return k(dst)


def _scatter_body(t_hbm, src_hbm, dst_hbm, out_hbm,
                  srcb, dstb, rows, zbuf, acc, sem):
    c = lax.axis_index("c")
    s = lax.axis_index("s")
    wid = s * NC + c
    row0 = s * ROWS_PER_TILE

    zeros16 = jnp.zeros((16,), jnp.float32)

    @pl.loop(0, ZR)
    def _(i):
        @pl.loop(0, H, step=16)
        def _(j):
            zbuf[i, pl.ds(j, 16)] = zeros16

    @pl.loop(0, ROWS_PER_TILE, step=ZR)
    def _(r):
        pltpu.sync_copy(zbuf, acc.at[pl.ds(row0 + r, ZR)])

    plsc.subcore_barrier()

    base = wid * EW

    @pl.loop(0, EW, step=CHUNK)
    def _(j):
        pltpu.async_copy(src_hbm.at[pl.ds(base + j, CHUNK)], srcb, sem).wait()
        pltpu.async_copy(dst_hbm.at[pl.ds(base + j, CHUNK)], dstb, sem).wait()
        pltpu.async_copy(t_hbm.at[srcb], rows, sem).wait()
        pltpu.sync_copy(rows, acc.at[dstb], add=True)

    plsc.subcore_barrier()

    pltpu.sync_copy(
        acc.at[pl.ds(row0, ROWS_PER_TILE)],
        out_hbm.at[c, pl.ds(row0, ROWS_PER_TILE)],
    )


def _sc_scatter(t, src, dst):
    k = pl.kernel(
        _scatter_body,
        out_type=jax.ShapeDtypeStruct((NC, N_PAD, H), jnp.float32),
        mesh=_mesh(),
        scratch_types=[
            pltpu.VMEM((CHUNK,), jnp.int32),
            pltpu.VMEM((CHUNK,), jnp.int32),
            pltpu.VMEM((CHUNK, H), jnp.float32),
            pltpu.VMEM((ZR, H), jnp.float32),
            pltpu.VMEM_SHARED((N_PAD, H), jnp.float32),
            pltpu.SemaphoreType.DMA,
        ],
        compiler_params=_sc_params(),
    )
    return k(t, src, dst)


def _pre_body(x_ref, win_ref, bin_ref, degp_ref, wc0_ref,
              h_ref, t_ref, dinv_ref):
    deg = jnp.sum(degp_ref[...], axis=0) + 1.0
    dinv = lax.rsqrt(jnp.maximum(deg, 1.0))
    h = jnp.maximum(
        jnp.dot(x_ref[...], win_ref[...], preferred_element_type=jnp.float32)
        + bin_ref[...][None, :],
        0.0,
    )
    t = jnp.dot(h, wc0_ref[...], preferred_element_type=jnp.float32)
    h_ref[...] = h
    t_ref[...] = t * dinv[:, None]
    dinv_ref[...] = dinv


def _tc_pre(x, w_in, b_in, degp, wc0):
    return pl.pallas_call(
        _pre_body,
        out_shape=[
            jax.ShapeDtypeStruct((N, H), jnp.float32),
            jax.ShapeDtypeStruct((N, H), jnp.float32),
            jax.ShapeDtypeStruct((N,), jnp.float32),
        ],
    )(x, w_in, b_in, degp, wc0)


def _post_body(has_res, is_final, *refs):
    if has_res and not is_final:
        sp_ref, t_ref, hres_ref, dinv_ref, g_ref, b_ref, bc_ref, wn_ref, \
            h_ref, tn_ref = refs
    elif not has_res and not is_final:
        sp_ref, t_ref, dinv_ref, g_ref, b_ref, bc_ref, wn_ref, \
            h_ref, tn_ref = refs
        hres_ref = None
    else:
        sp_ref, t_ref, hres_ref, dinv_ref, g_ref, b_ref, bc_ref, wout_ref, \
            bout_ref, out_ref = refs

    dinv = dinv_ref[...]
    t = t_ref[...]
    s = sp_ref[0, :N, :] + sp_ref[1, :N, :] + t
    agg = s * dinv[:, None] + bc_ref[...][None, :]
    mu = jnp.mean(agg, axis=0)
    ctr = agg - mu[None, :]
    var = jnp.mean(ctr * ctr, axis=0)
    hbn = ctr * lax.rsqrt(var + EPS) * g_ref[...][None, :] + b_ref[...][None, :]
    h = jnp.maximum(hbn, 0.0)
    if hres_ref is not None:
        h = h + hres_ref[...]
    if is_final:
        out_ref[...] = (
            jnp.dot(h, wout_ref[...], preferred_element_type=jnp.float32)
            + bout_ref[...][None, :]
        )
    else:
        h_ref[...] = h
        tn = jnp.dot(h, wn_ref[...], preferred_element_type=jnp.float32)
        tn_ref[...] = tn * dinv[:, None]


def _tc_post(sp, t, hres, dinv, g, b, bci, wnext, has_res):
    body = functools.partial(_post_body, has_res, False)
    args = (sp, t, hres, dinv, g, b, bci, wnext) if has_res else (
        sp, t, dinv, g, b, bci, wnext)
    return pl.pallas_call(
        body,
        out_shape=[
            jax.ShapeDtypeStruct((N, H), jnp.float32),
            jax.ShapeDtypeStruct((N, H), jnp.float32),
        ],
    )(*args)


def _tc_final(sp, t, hres, dinv, g, b, bci, w_out, b_out):
    body = functools.partial(_post_body, True, True)
    return pl.pallas_call(
        body,
        out_shape=jax.ShapeDtypeStruct((N, O_DIM), jnp.float32),
    )(sp, t, hres, dinv, g, b, bci, w_out, b_out)


def kernel(x, edge_index, W_in, b_in, Wc, bc, gamma, beta, W_out, b_out):
    src = edge_index[0]
    dst = edge_index[1]

    degp = _sc_degree(dst).reshape(NW, N)
    h, t, dinv = _tc_pre(x, W_in, b_in, degp, Wc[0])

    for i in range(L):
        sp = _sc_scatter(t, src, dst)
        if i < L - 1:
            h, t = _tc_post(sp, t, h, dinv, gamma[i], beta[i], bc[i],
                            Wc[i + 1], has_res=(i > 0))
        else:
            out = _tc_final(sp, t, h, dinv, gamma[i], beta[i], bc[i],
                            W_out, b_out)
    return out


# speedup vs baseline: 19.6830x; 1.6556x over previous
import dataclasses
import functools

import jax
import jax.numpy as jnp
from jax import lax
from jax.experimental import pallas as pl
from jax.experimental.pallas import tpu as pltpu
from jax.experimental.pallas import tpu_sc as plsc

N = 10000
E = 320000
D = 128
H = 128
O_DIM = 10
L = 3
EPS = 1e-5

NC = 2
NS = 16
NW = NC * NS
EW = E // NW
CHUNK = 40
NCH = EW // CHUNK
NBUF = 5
STAGE = 2000
N_PAD = 10112
ROWS_PER_TILE = N_PAD // NS
ZR = 24

_mesh = functools.partial(
    plsc.VectorSubcoreMesh, core_axis_name="c", subcore_axis_name="s"
)


def _sc_params():
    cp = pltpu.CompilerParams()
    if "needs_layout_passes" in pltpu.CompilerParams.__dataclass_fields__:
        cp = dataclasses.replace(cp, needs_layout_passes=False)
    return cp


def _deg_body(dst_hbm, out_hbm, idx_v, deg_v, sem):
    c = lax.axis_index("c")
    s = lax.axis_index("s")
    wid = s * NC + c

    zeros16 = jnp.zeros((16,), jnp.float32)
    ones16 = jnp.ones((16,), jnp.float32)

    @pl.loop(0, N, step=16)
    def _(i):
        deg_v[pl.ds(i, 16)] = zeros16

    base = wid * EW

    @pl.loop(0, EW, step=STAGE)
    def _(j):
        pltpu.async_copy(dst_hbm.at[pl.ds(base + j, STAGE)], idx_v, sem).wait()

        @pl.loop(0, STAGE, step=16)
        def _(k):
            idx = idx_v[pl.ds(k, 16)]
            plsc.addupdate_scatter(deg_v, [idx], ones16)

    pltpu.sync_copy(deg_v, out_hbm.at[pl.ds(wid * N, N)])


def _sc_degree(dst):
    k = pl.kernel(
        _deg_body,
        out_type=jax.ShapeDtypeStruct((NW * N,), jnp.float32),
        mesh=_mesh(),
        scratch_types=[
            pltpu.VMEM((STAGE,), jnp.int32),
            pltpu.VMEM((N,), jnp.float32),
            pltpu.SemaphoreType.DMA,
        ],
        compiler_params=_sc_params(),
    )
    return k(dst)


def _scatter_body(t_hbm, src_hbm, dst_hbm, out_hbm, *refs):
    srcb = refs[0:NBUF]
    dstb = refs[NBUF:2 * NBUF]
    rows = refs[2 * NBUF]
    zbuf = refs[2 * NBUF + 1]
    acc = refs[2 * NBUF + 2]
    sems = refs[2 * NBUF + 3:]

    c = lax.axis_index("c")
    s = lax.axis_index("s")
    wid = s * NC + c
    row0 = s * ROWS_PER_TILE

    zeros16 = jnp.zeros((16,), jnp.float32)

    @pl.loop(0, ZR)
    def _(i):
        @pl.loop(0, H, step=16)
        def _(j):
            zbuf[i, pl.ds(j, 16)] = zeros16

    @pl.loop(0, (ROWS_PER_TILE // ZR) * ZR, step=ZR)
    def _(r):
        pltpu.sync_copy(zbuf, acc.at[pl.ds(row0 + r, ZR)])
    _rem = ROWS_PER_TILE - (ROWS_PER_TILE // ZR) * ZR
    if _rem:
        pltpu.sync_copy(
            zbuf.at[pl.ds(0, _rem)],
            acc.at[pl.ds(row0 + (ROWS_PER_TILE // ZR) * ZR, _rem)],
        )

    plsc.subcore_barrier()

    base = wid * EW

    @pl.loop(0, EW, step=NBUF * CHUNK)
    def _(j):
        ih = []
        for b in range(NBUF):
            off = base + j + b * CHUNK
            ih.append((
                pltpu.async_copy(src_hbm.at[pl.ds(off, CHUNK)], srcb[b],
                                 sems[b]),
                pltpu.async_copy(dst_hbm.at[pl.ds(off, CHUNK)], dstb[b],
                                 sems[b]),
            ))
        gh = []
        for b in range(NBUF):
            ih[b][0].wait()
            ih[b][1].wait()
            gh.append(pltpu.async_copy(t_hbm.at[srcb[b]], rows.at[b],
                                       sems[b]))
        for b in range(NBUF):
            gh[b].wait()
            pltpu.sync_copy(rows.at[b], acc.at[dstb[b]], add=True)

    plsc.subcore_barrier()

    pltpu.sync_copy(
        acc.at[pl.ds(row0, ROWS_PER_TILE)],
        out_hbm.at[c, pl.ds(row0, ROWS_PER_TILE)],
    )


def _sc_scatter(t, src, dst):
    k = pl.kernel(
        _scatter_body,
        out_type=jax.ShapeDtypeStruct((NC, N_PAD, H), jnp.float32),
        mesh=_mesh(),
        scratch_types=(
            [pltpu.VMEM((CHUNK,), jnp.int32)] * NBUF
            + [pltpu.VMEM((CHUNK,), jnp.int32)] * NBUF
            + [
                pltpu.VMEM((NBUF, CHUNK, H), jnp.float32),
                pltpu.VMEM((ZR, H), jnp.float32),
                pltpu.VMEM_SHARED((N_PAD, H), jnp.float32),
            ]
            + [pltpu.SemaphoreType.DMA] * NBUF
        ),
        compiler_params=_sc_params(),
    )
    return k(t, src, dst)


def _pre_body(x_ref, win_ref, bin_ref, degp_ref, wc0_ref,
              h_ref, t_ref, dinv_ref):
    deg = jnp.sum(degp_ref[...], axis=0) + 1.0
    dinv = lax.rsqrt(jnp.maximum(deg, 1.0))
    h = jnp.maximum(
        jnp.dot(x_ref[...], win_ref[...], preferred_element_type=jnp.float32)
        + bin_ref[...][None, :],
        0.0,
    )
    t = jnp.dot(h, wc0_ref[...], preferred_element_type=jnp.float32)
    h_ref[...] = h
    t_ref[...] = t * dinv[:, None]
    dinv_ref[...] = dinv


def _tc_pre(x, w_in, b_in, degp, wc0):
    return pl.pallas_call(
        _pre_body,
        out_shape=[
            jax.ShapeDtypeStruct((N, H), jnp.float32),
            jax.ShapeDtypeStruct((N, H), jnp.float32),
            jax.ShapeDtypeStruct((N,), jnp.float32),
        ],
    )(x, w_in, b_in, degp, wc0)


def _post_body(has_res, is_final, *refs):
    if has_res and not is_final:
        sp_ref, t_ref, hres_ref, dinv_ref, g_ref, b_ref, bc_ref, wn_ref, \
            h_ref, tn_ref = refs
    elif not has_res and not is_final:
        sp_ref, t_ref, dinv_ref, g_ref, b_ref, bc_ref, wn_ref, \
            h_ref, tn_ref = refs
        hres_ref = None
    else:
        sp_ref, t_ref, hres_ref, dinv_ref, g_ref, b_ref, bc_ref, wout_ref, \
            bout_ref, out_ref = refs

    dinv = dinv_ref[...]
    t = t_ref[...]
    s = sp_ref[0, :N, :] + sp_ref[1, :N, :] + t
    agg = s * dinv[:, None] + bc_ref[...][None, :]
    mu = jnp.mean(agg, axis=0)
    ctr = agg - mu[None, :]
    var = jnp.mean(ctr * ctr, axis=0)
    hbn = ctr * lax.rsqrt(var + EPS) * g_ref[...][None, :] + b_ref[...][None, :]
    h = jnp.maximum(hbn, 0.0)
    if hres_ref is not None:
        h = h + hres_ref[...]
    if is_final:
        out_ref[...] = (
            jnp.dot(h, wout_ref[...], preferred_element_type=jnp.float32)
            + bout_ref[...][None, :]
        )
    else:
        h_ref[...] = h
        tn = jnp.dot(h, wn_ref[...], preferred_element_type=jnp.float32)
        tn_ref[...] = tn * dinv[:, None]


def _tc_post(sp, t, hres, dinv, g, b, bci, wnext, has_res):
    body = functools.partial(_post_body, has_res, False)
    args = (sp, t, hres, dinv, g, b, bci, wnext) if has_res else (
        sp, t, dinv, g, b, bci, wnext)
    return pl.pallas_call(
        body,
        out_shape=[
            jax.ShapeDtypeStruct((N, H), jnp.float32),
            jax.ShapeDtypeStruct((N, H), jnp.float32),
        ],
    )(*args)


def _tc_final(sp, t, hres, dinv, g, b, bci, w_out, b_out):
    body = functools.partial(_post_body, True, True)
    return pl.pallas_call(
        body,
        out_shape=jax.ShapeDtypeStruct((N, O_DIM), jnp.float32),
    )(sp, t, hres, dinv, g, b, bci, w_out, b_out)


def kernel(x, edge_index, W_in, b_in, Wc, bc, gamma, beta, W_out, b_out):
    src = edge_index[0]
    dst = edge_index[1]

    degp = _sc_degree(dst).reshape(NW, N)
    h, t, dinv = _tc_pre(x, W_in, b_in, degp, Wc[0])

    for i in range(L):
        sp = _sc_scatter(t, src, dst)
        if i < L - 1:
            h, t = _tc_post(sp, t, h, dinv, gamma[i], beta[i], bc[i],
                            Wc[i + 1], has_res=(i > 0))
        else:
            out = _tc_final(sp, t, h, dinv, gamma[i], beta[i], bc[i],
                            W_out, b_out)
    return out


# speedup vs baseline: 19.8914x; 1.0106x over previous
import dataclasses
import functools

import jax
import jax.numpy as jnp
from jax import lax
from jax.experimental import pallas as pl
from jax.experimental.pallas import tpu as pltpu
from jax.experimental.pallas import tpu_sc as plsc

N = 10000
E = 320000
D = 128
H = 128
O_DIM = 10
L = 3
EPS = 1e-5

NC = 2
NS = 16
NW = NC * NS
EW = E // NW
CHUNK = 40
NCH = EW // CHUNK
NBUF = 5
NGRP = NCH // NBUF
STAGE = 2000
N_PAD = 10112
ROWS_PER_TILE = N_PAD // NS
ZR = 24

_mesh = functools.partial(
    plsc.VectorSubcoreMesh, core_axis_name="c", subcore_axis_name="s"
)


def _sc_params():
    cp = pltpu.CompilerParams()
    if "needs_layout_passes" in pltpu.CompilerParams.__dataclass_fields__:
        cp = dataclasses.replace(cp, needs_layout_passes=False)
    return cp


def _deg_body(dst_hbm, out_hbm, idx_v, deg_v, sem):
    c = lax.axis_index("c")
    s = lax.axis_index("s")
    wid = s * NC + c

    zeros16 = jnp.zeros((16,), jnp.float32)
    ones16 = jnp.ones((16,), jnp.float32)

    @pl.loop(0, N, step=16)
    def _(i):
        deg_v[pl.ds(i, 16)] = zeros16

    base = wid * EW

    @pl.loop(0, EW, step=STAGE)
    def _(j):
        pltpu.async_copy(dst_hbm.at[pl.ds(base + j, STAGE)], idx_v, sem).wait()

        @pl.loop(0, STAGE, step=16)
        def _(k):
            idx = idx_v[pl.ds(k, 16)]
            plsc.addupdate_scatter(deg_v, [idx], ones16)

    pltpu.sync_copy(deg_v, out_hbm.at[pl.ds(wid * N, N)])


def _sc_degree(dst):
    k = pl.kernel(
        _deg_body,
        out_type=jax.ShapeDtypeStruct((NW * N,), jnp.float32),
        mesh=_mesh(),
        scratch_types=[
            pltpu.VMEM((STAGE,), jnp.int32),
            pltpu.VMEM((N,), jnp.float32),
            pltpu.SemaphoreType.DMA,
        ],
        compiler_params=_sc_params(),
    )
    return k(dst)


def _scatter_body(t_hbm, src_hbm, dst_hbm, out_hbm, *refs):
    srcb = refs[0:NBUF]
    dstb = refs[NBUF:2 * NBUF]
    rows = refs[2 * NBUF]
    zbuf = refs[2 * NBUF + 1]
    acc = refs[2 * NBUF + 2]
    isem = refs[2 * NBUF + 3:2 * NBUF + 3 + NBUF]
    gsem = refs[2 * NBUF + 3 + NBUF:2 * NBUF + 3 + 2 * NBUF]
    ssem = refs[2 * NBUF + 3 + 2 * NBUF:]

    c = lax.axis_index("c")
    s = lax.axis_index("s")
    wid = s * NC + c
    row0 = s * ROWS_PER_TILE
    base = wid * EW

    zeros16 = jnp.zeros((16,), jnp.float32)

    @pl.loop(0, ZR)
    def _(i):
        @pl.loop(0, H, step=16)
        def _(j):
            zbuf[i, pl.ds(j, 16)] = zeros16

    @pl.loop(0, (ROWS_PER_TILE // ZR) * ZR, step=ZR)
    def _(r):
        pltpu.sync_copy(zbuf, acc.at[pl.ds(row0 + r, ZR)])
    _rem = ROWS_PER_TILE - (ROWS_PER_TILE // ZR) * ZR
    if _rem:
        pltpu.sync_copy(
            zbuf.at[pl.ds(0, _rem)],
            acc.at[pl.ds(row0 + (ROWS_PER_TILE // ZR) * ZR, _rem)],
        )

    def load_idx_issue_gather(off, b):
        i1 = pltpu.async_copy(src_hbm.at[pl.ds(off, CHUNK)], srcb[b],
                              isem[b])
        i2 = pltpu.async_copy(dst_hbm.at[pl.ds(off, CHUNK)], dstb[b],
                              isem[b])
        i1.wait()
        i2.wait()
        return pltpu.async_copy(t_hbm.at[srcb[b]], rows.at[b], gsem[b])

    for b in range(NBUF):
        load_idx_issue_gather(base + b * CHUNK, b)

    plsc.subcore_barrier()

    def _drain_gather(b):
        pltpu.make_async_copy(t_hbm.at[srcb[b]], rows.at[b], gsem[b]).wait()

    def _process(g, issue_next):
        shs = []
        for b in range(NBUF):
            _drain_gather(b)
            shs.append(pltpu.async_copy(rows.at[b], acc.at[dstb[b]],
                                        ssem[b], add=True))
        for b in range(NBUF):
            shs[b].wait()
            if issue_next:
                load_idx_issue_gather(base + (g + 1) * NBUF * CHUNK
                                      + b * CHUNK, b)

    @pl.loop(0, NGRP - 1)
    def _(g):
        _process(g, True)

    _process(NGRP - 1, False)

    plsc.subcore_barrier()

    pltpu.sync_copy(
        acc.at[pl.ds(row0, ROWS_PER_TILE)],
        out_hbm.at[c, pl.ds(row0, ROWS_PER_TILE)],
    )


def _sc_scatter(t, src, dst):
    k = pl.kernel(
        _scatter_body,
        out_type=jax.ShapeDtypeStruct((NC, N_PAD, H), jnp.float32),
        mesh=_mesh(),
        scratch_types=(
            [pltpu.VMEM((CHUNK,), jnp.int32)] * NBUF
            + [pltpu.VMEM((CHUNK,), jnp.int32)] * NBUF
            + [
                pltpu.VMEM((NBUF, CHUNK, H), jnp.float32),
                pltpu.VMEM((ZR, H), jnp.float32),
                pltpu.VMEM_SHARED((N_PAD, H), jnp.float32),
            ]
            + [pltpu.SemaphoreType.DMA] * (3 * NBUF)
        ),
        compiler_params=_sc_params(),
    )
    return k(t, src, dst)


def _pre_body(x_ref, win_ref, bin_ref, degp_ref, wc0_ref,
              h_ref, t_ref, dinv_ref):
    deg = jnp.sum(degp_ref[...], axis=0) + 1.0
    dinv = lax.rsqrt(jnp.maximum(deg, 1.0))
    h = jnp.maximum(
        jnp.dot(x_ref[...], win_ref[...], preferred_element_type=jnp.float32)
        + bin_ref[...][None, :],
        0.0,
    )
    t = jnp.dot(h, wc0_ref[...], preferred_element_type=jnp.float32)
    h_ref[...] = h
    t_ref[...] = t * dinv[:, None]
    dinv_ref[...] = dinv


def _tc_pre(x, w_in, b_in, degp, wc0):
    return pl.pallas_call(
        _pre_body,
        out_shape=[
            jax.ShapeDtypeStruct((N, H), jnp.float32),
            jax.ShapeDtypeStruct((N, H), jnp.float32),
            jax.ShapeDtypeStruct((N,), jnp.float32),
        ],
    )(x, w_in, b_in, degp, wc0)


def _post_body(has_res, is_final, *refs):
    if has_res and not is_final:
        sp_ref, t_ref, hres_ref, dinv_ref, g_ref, b_ref, bc_ref, wn_ref, \
            h_ref, tn_ref = refs
    elif not has_res and not is_final:
        sp_ref, t_ref, dinv_ref, g_ref, b_ref, bc_ref, wn_ref, \
            h_ref, tn_ref = refs
        hres_ref = None
    else:
        sp_ref, t_ref, hres_ref, dinv_ref, g_ref, b_ref, bc_ref, wout_ref, \
            bout_ref, out_ref = refs

    dinv = dinv_ref[...]
    t = t_ref[...]
    s = sp_ref[0, :N, :] + sp_ref[1, :N, :] + t
    agg = s * dinv[:, None] + bc_ref[...][None, :]
    mu = jnp.mean(agg, axis=0)
    ctr = agg - mu[None, :]
    var = jnp.mean(ctr * ctr, axis=0)
    hbn = ctr * lax.rsqrt(var + EPS) * g_ref[...][None, :] + b_ref[...][None, :]
    h = jnp.maximum(hbn, 0.0)
    if hres_ref is not None:
        h = h + hres_ref[...]
    if is_final:
        out_ref[...] = (
            jnp.dot(h, wout_ref[...], preferred_element_type=jnp.float32)
            + bout_ref[...][None, :]
        )
    else:
        h_ref[...] = h
        tn = jnp.dot(h, wn_ref[...], preferred_element_type=jnp.float32)
        tn_ref[...] = tn * dinv[:, None]


def _tc_post(sp, t, hres, dinv, g, b, bci, wnext, has_res):
    body = functools.partial(_post_body, has_res, False)
    args = (sp, t, hres, dinv, g, b, bci, wnext) if has_res else (
        sp, t, dinv, g, b, bci, wnext)
    return pl.pallas_call(
        body,
        out_shape=[
            jax.ShapeDtypeStruct((N, H), jnp.float32),
            jax.ShapeDtypeStruct((N, H), jnp.float32),
        ],
    )(*args)


def _tc_final(sp, t, hres, dinv, g, b, bci, w_out, b_out):
    body = functools.partial(_post_body, True, True)
    return pl.pallas_call(
        body,
        out_shape=jax.ShapeDtypeStruct((N, O_DIM), jnp.float32),
    )(sp, t, hres, dinv, g, b, bci, w_out, b_out)


def kernel(x, edge_index, W_in, b_in, Wc, bc, gamma, beta, W_out, b_out):
    src = edge_index[0]
    dst = edge_index[1]

    degp = _sc_degree(dst).reshape(NW, N)
    h, t, dinv = _tc_pre(x, W_in, b_in, degp, Wc[0])

    for i in range(L):
        sp = _sc_scatter(t, src, dst)
        if i < L - 1:
            h, t = _tc_post(sp, t, h, dinv, gamma[i], beta[i], bc[i],
                            Wc[i + 1], has_res=(i > 0))
        else:
            out = _tc_final(sp, t, h, dinv, gamma[i], beta[i], bc[i],
                            W_out, b_out)
    return out


# speedup vs baseline: 19.8988x; 1.0004x over previous
import functools

import jax
import jax.numpy as jnp
from jax import lax
from jax.experimental import pallas as pl
from jax.experimental.pallas import tpu as pltpu
from jax.experimental.pallas import tpu_sc as plsc

N = 10000
E = 320000
D = 128
H = 128
O_DIM = 10
L = 3
EPS = 1e-5

NC = 2
NS = 16
NW = NC * NS
EW = E // NW
CHUNK = 40
NCH = EW // CHUNK
NBUF = 5
NGRP = NCH // NBUF
STAGE = 2000
N_PAD = 10112
ROWS_PER_TILE = N_PAD // NS
ZR = 24

_mesh = functools.partial(
    plsc.VectorSubcoreMesh, core_axis_name="c", subcore_axis_name="s"
)


def _sc_params():
    return pltpu.CompilerParams(needs_layout_passes=False)


def _deg_body(dst_hbm, out_hbm, idx_v, deg_v, sem):
    c = lax.axis_index("c")
    s = lax.axis_index("s")
    wid = s * NC + c

    zeros16 = jnp.zeros((16,), jnp.float32)
    ones16 = jnp.ones((16,), jnp.float32)

    @pl.loop(0, N, step=16)
    def _(i):
        deg_v[pl.ds(i, 16)] = zeros16

    base = wid * EW

    @pl.loop(0, EW, step=STAGE)
    def _(j):
        pltpu.async_copy(dst_hbm.at[pl.ds(base + j, STAGE)], idx_v, sem).wait()

        @pl.loop(0, STAGE, step=16)
        def _(k):
            idx = idx_v[pl.ds(k, 16)]
            plsc.addupdate_scatter(deg_v, [idx], ones16)

    pltpu.sync_copy(deg_v, out_hbm.at[pl.ds(wid * N, N)])


def _sc_degree(dst):
    k = pl.kernel(
        _deg_body,
        out_type=jax.ShapeDtypeStruct((NW * N,), jnp.float32),
        mesh=_mesh(),
        scratch_types=[
            pltpu.VMEM((STAGE,), jnp.int32),
            pltpu.VMEM((N,), jnp.float32),
            pltpu.SemaphoreType.DMA,
        ],
        compiler_params=_sc_params(),
    )
    return k(dst)


def _scatter_body(t_hbm, src_hbm, dst_hbm, out_hbm, *refs):
    srcb = refs[0:NBUF]
    dstb = refs[NBUF:2 * NBUF]
    rows = refs[2 * NBUF]
    zbuf = refs[2 * NBUF + 1]
    acc = refs[2 * NBUF + 2]
    isem = refs[2 * NBUF + 3:2 * NBUF + 3 + NBUF]
    gsem = refs[2 * NBUF + 3 + NBUF:2 * NBUF + 3 + 2 * NBUF]
    ssem = refs[2 * NBUF + 3 + 2 * NBUF:]

    c = lax.axis_index("c")
    s = lax.axis_index("s")
    wid = s * NC + c
    row0 = s * ROWS_PER_TILE
    base = wid * EW

    zeros16 = jnp.zeros((16,), jnp.float32)

    @pl.loop(0, ZR)
    def _(i):
        @pl.loop(0, H, step=16)
        def _(j):
            zbuf[i, pl.ds(j, 16)] = zeros16

    @pl.loop(0, (ROWS_PER_TILE // ZR) * ZR, step=ZR)
    def _(r):
        pltpu.sync_copy(zbuf, acc.at[pl.ds(row0 + r, ZR)])
    _rem = ROWS_PER_TILE - (ROWS_PER_TILE // ZR) * ZR
    if _rem:
        pltpu.sync_copy(
            zbuf.at[pl.ds(0, _rem)],
            acc.at[pl.ds(row0 + (ROWS_PER_TILE // ZR) * ZR, _rem)],
        )

    def load_idx_issue_gather(off, b):
        i1 = pltpu.async_copy(src_hbm.at[pl.ds(off, CHUNK)], srcb[b],
                              isem[b])
        i2 = pltpu.async_copy(dst_hbm.at[pl.ds(off, CHUNK)], dstb[b],
                              isem[b])
        i1.wait()
        i2.wait()
        return pltpu.async_copy(t_hbm.at[srcb[b]], rows.at[b], gsem[b])

    for b in range(NBUF):
        load_idx_issue_gather(base + b * CHUNK, b)

    plsc.subcore_barrier()

    def _drain_gather(b):
        pltpu.make_async_copy(t_hbm.at[srcb[b]], rows.at[b], gsem[b]).wait()

    def _process(g, issue_next):
        shs = []
        for b in range(NBUF):
            _drain_gather(b)
            shs.append(pltpu.async_copy(rows.at[b], acc.at[dstb[b]],
                                        ssem[b], add=True))
        for b in range(NBUF):
            shs[b].wait()
            if issue_next:
                load_idx_issue_gather(base + (g + 1) * NBUF * CHUNK
                                      + b * CHUNK, b)

    @pl.loop(0, NGRP - 1)
    def _(g):
        _process(g, True)

    _process(NGRP - 1, False)

    plsc.subcore_barrier()

    pltpu.sync_copy(
        acc.at[pl.ds(row0, ROWS_PER_TILE)],
        out_hbm.at[c, pl.ds(row0, ROWS_PER_TILE)],
    )


def _sc_scatter(t, src, dst):
    k = pl.kernel(
        _scatter_body,
        out_type=jax.ShapeDtypeStruct((NC, N_PAD, H), jnp.float32),
        mesh=_mesh(),
        scratch_types=(
            [pltpu.VMEM((CHUNK,), jnp.int32)] * NBUF
            + [pltpu.VMEM((CHUNK,), jnp.int32)] * NBUF
            + [
                pltpu.VMEM((NBUF, CHUNK, H), jnp.float32),
                pltpu.VMEM((ZR, H), jnp.float32),
                pltpu.VMEM_SHARED((N_PAD, H), jnp.float32),
            ]
            + [pltpu.SemaphoreType.DMA] * (3 * NBUF)
        ),
        compiler_params=_sc_params(),
    )
    return k(t, src, dst)


def _pre_body(x_ref, win_ref, bin_ref, degp_ref, wc0_ref,
              h_ref, t_ref, dinv_ref):
    deg = jnp.sum(degp_ref[...], axis=0) + 1.0
    dinv = lax.rsqrt(jnp.maximum(deg, 1.0))
    h = jnp.maximum(
        jnp.dot(x_ref[...], win_ref[...], preferred_element_type=jnp.float32)
        + bin_ref[...][None, :],
        0.0,
    )
    t = jnp.dot(h, wc0_ref[...], preferred_element_type=jnp.float32)
    h_ref[...] = h
    t_ref[...] = t * dinv[:, None]
    dinv_ref[...] = dinv


def _tc_pre(x, w_in, b_in, degp, wc0):
    return pl.pallas_call(
        _pre_body,
        out_shape=[
            jax.ShapeDtypeStruct((N, H), jnp.float32),
            jax.ShapeDtypeStruct((N, H), jnp.float32),
            jax.ShapeDtypeStruct((N,), jnp.float32),
        ],
    )(x, w_in, b_in, degp, wc0)


def _post_body(has_res, is_final, *refs):
    if has_res and not is_final:
        sp_ref, t_ref, hres_ref, dinv_ref, g_ref, b_ref, bc_ref, wn_ref, \
            h_ref, tn_ref = refs
    elif not has_res and not is_final:
        sp_ref, t_ref, dinv_ref, g_ref, b_ref, bc_ref, wn_ref, \
            h_ref, tn_ref = refs
        hres_ref = None
    else:
        sp_ref, t_ref, hres_ref, dinv_ref, g_ref, b_ref, bc_ref, wout_ref, \
            bout_ref, out_ref = refs

    dinv = dinv_ref[...]
    t = t_ref[...]
    s = sp_ref[0, :N, :] + sp_ref[1, :N, :] + t
    agg = s * dinv[:, None] + bc_ref[...][None, :]
    mu = jnp.mean(agg, axis=0)
    ctr = agg - mu[None, :]
    var = jnp.mean(ctr * ctr, axis=0)
    hbn = ctr * lax.rsqrt(var + EPS) * g_ref[...][None, :] + b_ref[...][None, :]
    h = jnp.maximum(hbn, 0.0)
    if hres_ref is not None:
        h = h + hres_ref[...]
    if is_final:
        out_ref[...] = (
            jnp.dot(h, wout_ref[...], preferred_element_type=jnp.float32)
            + bout_ref[...][None, :]
        )
    else:
        h_ref[...] = h
        tn = jnp.dot(h, wn_ref[...], preferred_element_type=jnp.float32)
        tn_ref[...] = tn * dinv[:, None]


def _tc_post(sp, t, hres, dinv, g, b, bci, wnext, has_res):
    body = functools.partial(_post_body, has_res, False)
    args = (sp, t, hres, dinv, g, b, bci, wnext) if has_res else (
        sp, t, dinv, g, b, bci, wnext)
    return pl.pallas_call(
        body,
        out_shape=[
            jax.ShapeDtypeStruct((N, H), jnp.float32),
            jax.ShapeDtypeStruct((N, H), jnp.float32),
        ],
    )(*args)


def _tc_final(sp, t, hres, dinv, g, b, bci, w_out, b_out):
    body = functools.partial(_post_body, True, True)
    return pl.pallas_call(
        body,
        out_shape=jax.ShapeDtypeStruct((N, O_DIM), jnp.float32),
    )(sp, t, hres, dinv, g, b, bci, w_out, b_out)


def kernel(x, edge_index, W_in, b_in, Wc, bc, gamma, beta, W_out, b_out):
    src = edge_index[0]
    dst = edge_index[1]

    degp = _sc_degree(dst).reshape(NW, N)
    h, t, dinv = _tc_pre(x, W_in, b_in, degp, Wc[0])

    for i in range(L):
        sp = _sc_scatter(t, src, dst)
        if i < L - 1:
            h, t = _tc_post(sp, t, h, dinv, gamma[i], beta[i], bc[i],
                            Wc[i + 1], has_res=(i > 0))
        else:
            out = _tc_final(sp, t, h, dinv, gamma[i], beta[i], bc[i],
                            W_out, b_out)
    return out
